# baseline (speedup 1.0000x reference)
import functools
import jax
import jax.numpy as jnp
from jax import lax
from jax.experimental import pallas as pl
from jax.experimental.pallas import tpu as pltpu
from jax.experimental.pallas import tpu_sc as plsc

N_NODES_K = 10000
N_EDGES_K = 160000
D_K = 256
H_K = D_K // 2
B_K = 128
NSUB = 16
L = 16
T_BLK = 80
NBLK_PAD = T_BLK * NSUB
E_PAD = NBLK_PAD * B_K
G_CH = 8
N_CH = T_BLK // G_CH
NSTG = 64
TRASH = N_NODES_K
ACC_ROWS = N_NODES_K + 8
ROWS_PER_SUB = 624

_mesh = plsc.VectorSubcoreMesh(core_axis_name="c", subcore_axis_name="s")


@functools.partial(
    pl.kernel,
    out_type=jax.ShapeDtypeStruct((2, N_NODES_K, H_K), jnp.float32),
    mesh=_mesh,
    scratch_types=[
        pltpu.VMEM((2, G_CH, 2, B_K), jnp.int32),
        pltpu.VMEM((2, G_CH, B_K), jnp.float32),
        pltpu.VMEM((2, B_K, H_K), jnp.float32),
        pltpu.VMEM((NSTG, H_K), jnp.float32),
        pltpu.VMEM((NSTG,), jnp.int32),
        pltpu.VMEM_SHARED((ACC_ROWS, H_K), jnp.float32),
        pltpu.SemaphoreType.DMA,
        pltpu.SemaphoreType.DMA,
        pltpu.SemaphoreType.DMA,
    ],
    compiler_params=pltpu.CompilerParams(needs_layout_passes=False),
)
def _spmm_sc(idx_h, val_h, oa_h, ob_h, out_h,
             rcb, vb, mb, stg, srw, acc, sem_i, sg0, sg1):
    c = lax.axis_index("c")
    s = lax.axis_index("s")
    sgs = [sg0, sg1]
    zeros16 = jnp.zeros((L,), jnp.float32)
    trash16 = jnp.full((L,), TRASH, jnp.int32)
    lane0 = lax.iota(jnp.int32, L) == 0
    r0 = s * ROWS_PER_SUB
    blk0 = s * T_BLK

    @pl.loop(0, B_K)
    def _(r):
        for jj in range(H_K // L):
            mb[0, r, pl.ds(jj * L, L)] = zeros16

    for kk in range(4):
        pltpu.sync_copy(mb.at[0], acc.at[pl.ds(r0 + kk * B_K, B_K), :])

    @pl.when(s < NSUB - 1)
    def _():
        pltpu.sync_copy(mb.at[0].at[pl.ds(0, 112), :],
                        acc.at[pl.ds(r0 + 4 * B_K, 112), :])

    @pl.when(s == NSUB - 1)
    def _():
        pltpu.sync_copy(mb.at[0], acc.at[pl.ds(r0 + 4 * B_K, B_K), :])

    for g4 in range(NSTG // L):
        srw[pl.ds(g4 * L, L)] = trash16

    plsc.subcore_barrier()

    def chunk_load(ch):
        pltpu.async_copy(
            idx_h.at[pl.ds((blk0 + ch * G_CH) * 1, G_CH), :, :],
            rcb.at[lax.rem(ch, 2)], sem_i)
        pltpu.async_copy(
            val_h.at[pl.ds(blk0 + ch * G_CH, G_CH), :],
            vb.at[lax.rem(ch, 2)], sem_i)

    def chunk_wait():
        pltpu.make_async_copy(idx_h.at[pl.ds(0, G_CH), :, :],
                              rcb.at[0], sem_i).wait()
        pltpu.make_async_copy(val_h.at[pl.ds(0, G_CH), :],
                              vb.at[0], sem_i).wait()

    def gather_issue(t, j):
        ch = t // G_CH
        colref = rcb.at[lax.rem(ch, 2), lax.rem(t, G_CH), 1]

        @pl.when(c == 0)
        def _():
            pltpu.async_copy(oa_h.at[colref], mb.at[j], sgs[j])

        @pl.when(c == 1)
        def _():
            pltpu.async_copy(ob_h.at[colref], mb.at[j], sgs[j])

    def gather_wait(j):
        pltpu.make_async_copy(oa_h.at[pl.ds(0, B_K), :],
                              mb.at[j], sgs[j]).wait()

    chunk_load(0)
    chunk_wait()
    gather_issue(0, 0)

    def block_body(t, j, carry):
        ch = t // G_CH
        jloc = lax.rem(t, G_CH)
        chpar = lax.rem(ch, 2)

        @pl.when(jnp.logical_and(jloc == 0, ch + 1 < N_CH))
        def _():
            chunk_load(ch + 1)

        @pl.when(jnp.logical_and(jloc == G_CH - 1, ch + 1 < N_CH))
        def _():
            chunk_wait()

        @pl.when(t + 1 < T_BLK)
        def _():
            gather_issue(t + 1, 1 - j)

        gather_wait(j)

        @pl.loop(0, B_K // L, init_carry=carry)
        def inner(g, icarry):
            prev, seg = icarry[0], icarry[1]
            accs = list(icarry[2:])
            rows16 = rcb[chpar, jloc, 0, pl.ds(g * L, L)]
            vals16 = vb[chpar, jloc, pl.ds(g * L, L)]
            for i in range(L):
                rid = rows16[i]
                vspl = jnp.take_along_axis(
                    vals16, jnp.full((L,), i, jnp.int32), axis=0)
                changed = rid != prev
                seg = seg + changed.astype(jnp.int32)

                @pl.when(jnp.logical_and(
                    changed,
                    jnp.logical_and(seg > 0,
                                    jnp.bitwise_and(seg, NSTG - 1) == 0)))
                def _():
                    pltpu.sync_copy(stg, acc.at[srw], add=True)
                    for g4 in range(NSTG // L):
                        srw[pl.ds(g4 * L, L)] = trash16

                slot = jnp.bitwise_and(seg, NSTG - 1)
                keep = jnp.where(changed, 0.0, 1.0).astype(jnp.float32)
                e = g * L + i
                for jj in range(H_K // L):
                    gv = mb[j, e, pl.ds(jj * L, L)]
                    accs[jj] = gv * vspl + accs[jj] * keep
                    stg[slot, pl.ds(jj * L, L)] = accs[jj]
                plsc.store_scatter(
                    srw, [jnp.broadcast_to(slot, (L,))],
                    jnp.broadcast_to(rid, (L,)), mask=lane0)
                prev = rid
            return (prev, seg, *accs)

        return inner

    carry = (jnp.int32(-1), jnp.int32(-1)) + (zeros16,) * (H_K // L)

    @pl.loop(0, T_BLK // 2, init_carry=carry)
    def final_carry(q, qcarry):
        qcarry = block_body(2 * q, 0, qcarry)
        qcarry = block_body(2 * q + 1, 1, qcarry)
        return qcarry

    pltpu.sync_copy(stg, acc.at[srw], add=True)

    plsc.subcore_barrier()

    for kk in range(4):
        pltpu.sync_copy(acc.at[pl.ds(r0 + kk * B_K, B_K), :],
                        out_h.at[c, pl.ds(r0 + kk * B_K, B_K), :])

    @pl.when(s < NSUB - 1)
    def _():
        pltpu.sync_copy(acc.at[pl.ds(r0 + 4 * B_K, 112), :],
                        out_h.at[c, pl.ds(r0 + 4 * B_K, 112), :])

    @pl.when(s == NSUB - 1)
    def _():
        pltpu.sync_copy(acc.at[pl.ds(r0 + 4 * B_K, B_K), :],
                        out_h.at[c, pl.ds(r0 + 4 * B_K, B_K), :])


def kernel(row, col, value, other):
    pad = E_PAD - N_EDGES_K
    zi = jnp.zeros((pad,), jnp.int32)
    row2 = jnp.concatenate([row, zi]).reshape(NBLK_PAD, 1, B_K)
    col2 = jnp.concatenate([col, zi]).reshape(NBLK_PAD, 1, B_K)
    idxp = jnp.concatenate([row2, col2], axis=1)
    val2 = jnp.concatenate(
        [value, jnp.zeros((pad,), jnp.float32)]).reshape(NBLK_PAD, B_K)
    oa = other[:, :H_K]
    ob = other[:, H_K:]
    out2 = _spmm_sc(idxp, val2, oa, ob)
    return out2.transpose(1, 0, 2).reshape(N_NODES_K, D_K)


# speedup vs baseline: 1.1767x; 1.1767x over previous
import functools
import jax
import jax.numpy as jnp
from jax import lax
from jax.experimental import pallas as pl
from jax.experimental.pallas import tpu as pltpu
from jax.experimental.pallas import tpu_sc as plsc

N_NODES_K = 10000
N_EDGES_K = 160000
D_K = 256
H_K = D_K // 2
B_K = 128
NSUB = 16
L = 16
T_BLK = 80
NBLK_PAD = T_BLK * NSUB
E_PAD = NBLK_PAD * B_K
G_CH = 8
N_CH = T_BLK // G_CH
NSTG = 64
TRASH = N_NODES_K
ACC_ROWS = N_NODES_K + 8
ROWS_PER_SUB = 624

_mesh = plsc.VectorSubcoreMesh(core_axis_name="c", subcore_axis_name="s")


@functools.partial(
    pl.kernel,
    out_type=jax.ShapeDtypeStruct((2, N_NODES_K, H_K), jnp.float32),
    mesh=_mesh,
    scratch_types=[
        pltpu.VMEM((2, G_CH, 2, B_K), jnp.int32),
        pltpu.VMEM((2, G_CH, B_K), jnp.float32),
        pltpu.VMEM((2, B_K, H_K), jnp.float32),
        pltpu.VMEM((NSTG, H_K), jnp.float32),
        pltpu.VMEM((NSTG,), jnp.int32),
        pltpu.VMEM_SHARED((ACC_ROWS, H_K), jnp.float32),
        pltpu.SemaphoreType.DMA,
        pltpu.SemaphoreType.DMA,
        pltpu.SemaphoreType.DMA,
    ],
    compiler_params=pltpu.CompilerParams(needs_layout_passes=False),
)
def _spmm_sc(idx_h, val_h, oa_h, ob_h, out_h,
             rcb, vb, mb, stg, srw, acc, sem_i, sg0, sg1):
    c = lax.axis_index("c")
    s = lax.axis_index("s")
    sgs = [sg0, sg1]
    zeros16 = jnp.zeros((L,), jnp.float32)
    trash16 = jnp.full((L,), TRASH, jnp.int32)
    lane0 = lax.iota(jnp.int32, L) == 0
    shift_idx = jnp.maximum(lax.iota(jnp.int32, L) - 1, 0)
    r0 = s * ROWS_PER_SUB
    blk0 = s * T_BLK

    @pl.loop(0, B_K)
    def _(r):
        for jj in range(H_K // L):
            mb[0, r, pl.ds(jj * L, L)] = zeros16

    for kk in range(4):
        pltpu.sync_copy(mb.at[0], acc.at[pl.ds(r0 + kk * B_K, B_K), :])

    @pl.when(s < NSUB - 1)
    def _():
        pltpu.sync_copy(mb.at[0].at[pl.ds(0, 112), :],
                        acc.at[pl.ds(r0 + 4 * B_K, 112), :])

    @pl.when(s == NSUB - 1)
    def _():
        pltpu.sync_copy(mb.at[0], acc.at[pl.ds(r0 + 4 * B_K, B_K), :])

    def stg_reset():
        @pl.loop(0, NSTG)
        def _(r):
            for jj in range(H_K // L):
                stg[r, pl.ds(jj * L, L)] = zeros16

        for g4 in range(NSTG // L):
            srw[pl.ds(g4 * L, L)] = trash16

    stg_reset()
    plsc.subcore_barrier()

    def chunk_load(ch):
        chp = lax.rem(ch, 2)
        pltpu.async_copy(idx_h.at[pl.ds(blk0 + ch * G_CH, G_CH), :, :],
                         rcb.at[chp], sem_i)
        pltpu.async_copy(val_h.at[pl.ds(blk0 + ch * G_CH, G_CH), :],
                         vb.at[chp], sem_i)

    def chunk_wait():
        pltpu.make_async_copy(idx_h.at[pl.ds(0, G_CH), :, :],
                              rcb.at[0], sem_i).wait()
        pltpu.make_async_copy(val_h.at[pl.ds(0, G_CH), :],
                              vb.at[0], sem_i).wait()

    def gather_issue(t, j):
        ch = t // G_CH
        colref = rcb.at[lax.rem(ch, 2), lax.rem(t, G_CH), 1]

        @pl.when(c == 0)
        def _():
            pltpu.async_copy(oa_h.at[colref], mb.at[j], sgs[j])

        @pl.when(c == 1)
        def _():
            pltpu.async_copy(ob_h.at[colref], mb.at[j], sgs[j])

    def gather_wait(j):
        pltpu.make_async_copy(oa_h.at[pl.ds(0, B_K), :],
                              mb.at[j], sgs[j]).wait()

    chunk_load(0)
    chunk_wait()
    gather_issue(0, 0)

    def block_body(t, j, carry):
        ch = t // G_CH
        jloc = lax.rem(t, G_CH)
        chpar = lax.rem(ch, 2)

        @pl.when(jnp.logical_and(jloc == 0, ch + 1 < N_CH))
        def _():
            chunk_load(ch + 1)

        @pl.when(jnp.logical_and(jloc == G_CH - 1, ch + 1 < N_CH))
        def _():
            chunk_wait()

        @pl.when(t + 1 < T_BLK)
        def _():
            gather_issue(t + 1, 1 - j)

        gather_wait(j)

        @pl.loop(0, B_K // L, init_carry=carry)
        def inner(g, icarry):
            prev16, seg, wb = icarry

            full = seg - wb >= NSTG - L

            @pl.when(full)
            def _():
                pltpu.sync_copy(stg, acc.at[srw], add=True)
                stg_reset()
                plsc.store_scatter(srw, [jnp.zeros((L,), jnp.int32)],
                                   prev16, mask=lane0)

            wb = jnp.where(full, seg, wb)

            rows16 = rcb[chpar, jloc, 0, pl.ds(g * L, L)]
            vals16 = vb[chpar, jloc, pl.ds(g * L, L)]
            shifted = jnp.take_along_axis(rows16, shift_idx, axis=0)
            prevs16 = jnp.where(lane0, prev16, shifted)
            changed16 = rows16 != prevs16
            seg16 = seg + plsc.cumsum(changed16.astype(jnp.int32))
            slot16 = seg16 - wb
            plsc.store_scatter(srw, [slot16], rows16, mask=changed16)

            for i in range(L):
                vspl = jnp.take_along_axis(
                    vals16, jnp.full((L,), i, jnp.int32), axis=0)
                sloti = slot16[i]
                e = g * L + i
                for jj in range(H_K // L):
                    gv = mb[j, e, pl.ds(jj * L, L)]
                    plsc.addupdate(stg.at[sloti, pl.ds(jj * L, L)],
                                   gv * vspl)

            prev16_new = jnp.take_along_axis(
                rows16, jnp.full((L,), L - 1, jnp.int32), axis=0)
            return (prev16_new, seg16[L - 1], wb)

        return inner

    carry = (jnp.full((L,), -1, jnp.int32), jnp.int32(-1), jnp.int32(0))

    @pl.loop(0, T_BLK // 2, init_carry=carry)
    def final_carry(q, qcarry):
        qcarry = block_body(2 * q, 0, qcarry)
        qcarry = block_body(2 * q + 1, 1, qcarry)
        return qcarry

    pltpu.sync_copy(stg, acc.at[srw], add=True)

    plsc.subcore_barrier()

    for kk in range(4):
        pltpu.sync_copy(acc.at[pl.ds(r0 + kk * B_K, B_K), :],
                        out_h.at[c, pl.ds(r0 + kk * B_K, B_K), :])

    @pl.when(s < NSUB - 1)
    def _():
        pltpu.sync_copy(acc.at[pl.ds(r0 + 4 * B_K, 112), :],
                        out_h.at[c, pl.ds(r0 + 4 * B_K, 112), :])

    @pl.when(s == NSUB - 1)
    def _():
        pltpu.sync_copy(acc.at[pl.ds(r0 + 4 * B_K, B_K), :],
                        out_h.at[c, pl.ds(r0 + 4 * B_K, B_K), :])


def kernel(row, col, value, other):
    pad = E_PAD - N_EDGES_K
    zi = jnp.zeros((pad,), jnp.int32)
    row2 = jnp.concatenate([row, zi]).reshape(NBLK_PAD, 1, B_K)
    col2 = jnp.concatenate([col, zi]).reshape(NBLK_PAD, 1, B_K)
    idxp = jnp.concatenate([row2, col2], axis=1)
    val2 = jnp.concatenate(
        [value, jnp.zeros((pad,), jnp.float32)]).reshape(NBLK_PAD, B_K)
    oa = other[:, :H_K]
    ob = other[:, H_K:]
    out2 = _spmm_sc(idxp, val2, oa, ob)
    return out2.transpose(1, 0, 2).reshape(N_NODES_K, D_K)
